# Initial kernel scaffold; baseline (speedup 1.0000x reference)
import jax
import jax.numpy as jnp
from jax.experimental import pallas as pl


def kernel(x, edge_index, batch, W1, a_src1, a_dst1, b1, W2, a_src2, a_dst2, b2):
    raise NotImplementedError("write your pallas kernel here")


# speedup vs baseline: 20.7876x; 20.7876x over previous
import functools
import jax
import jax.numpy as jnp
from jax import lax
from jax.experimental import pallas as pl
from jax.experimental.pallas import tpu as pltpu
import jax.experimental.pallas.tpu_sc as plsc

H = 8
HP = 16
HID = 16
OUTC = 128
NG = 128
NC = 2
NS = 16
NW = NC * NS
C = 80
C2 = 32

f32 = jnp.float32


def _lrelu(v):
    return jnp.maximum(v, 0.2 * v)


def _elu(v):
    return jnp.where(v > 0, v, jnp.exp(v) - 1.0)


def _copy_shared_out(sh_ref, out_ref, c, s, N):
    nm = (N // NS) & ~7
    tail = N - NS * nm
    off = s * nm
    pltpu.sync_copy(sh_ref.at[pl.ds(off, nm)], out_ref.at[c, pl.ds(off, nm)])
    if tail:
        @pl.when(s == NS - 1)
        def _():
            pltpu.sync_copy(sh_ref.at[pl.ds(NS * nm, tail)],
                            out_ref.at[c, pl.ds(NS * nm, tail)])


def _k0_body(x_ref, w1_ref, a1s_ref, a1d_ref, h1p_ref, as1_ref, ad1_ref):
    h = jnp.dot(x_ref[...], w1_ref[...], preferred_element_type=f32)
    h1p_ref[...] = h
    as1_ref[...] = jnp.dot(h, a1s_ref[...], preferred_element_type=f32)
    ad1_ref[...] = jnp.dot(h, a1d_ref[...], preferred_element_type=f32)


def _k1_body(s1_ref, d1_ref, h1p_ref, as1_ref, ad1_ref, b1_ref, w2_ref,
             ws2_ref, wd2_ref, rep_ref, h2p_ref, as2_ref, ad2_ref):
    wl1 = jnp.exp(_lrelu((as1_ref[...] + ad1_ref[...])[:, :H]))
    den8 = d1_ref[0][:, :H] + d1_ref[1][:, :H] + wl1
    den = jnp.dot(den8, rep_ref[...], preferred_element_type=f32)
    num = s1_ref[0] + s1_ref[1] + jnp.dot(
        wl1, rep_ref[...], preferred_element_type=f32) * h1p_ref[...]
    h1 = _elu(num / den + b1_ref[...])
    h2p_ref[...] = jnp.dot(h1, w2_ref[...], preferred_element_type=f32)
    as2_ref[...] = jnp.dot(h1, ws2_ref[...], preferred_element_type=f32)
    ad2_ref[...] = jnp.dot(h1, wd2_ref[...], preferred_element_type=f32)


def _k2_body(as2_ref, ad2_ref, d2_ref, dt_ref):
    wl2 = jnp.exp(_lrelu(as2_ref[...] + ad2_ref[...]))
    dsum = d2_ref[0] + d2_ref[1]
    pad = jnp.zeros((dsum.shape[0], wl2.shape[1] - HP), f32)
    dt_ref[...] = wl2 + jnp.concatenate([dsum, pad], axis=1)


def _k3_body(s2_ref, h2p_ref, as2_ref, ad2_ref, dt_ref, b2_ref, batch_ref,
             rep2_ref, sum2_ref, out_ref, cnt_ref):
    i = pl.program_id(0)
    wl2 = jnp.exp(_lrelu((as2_ref[...] + ad2_ref[...])[:, :H]))
    coef = wl2 / (8.0 * dt_ref[...][:, :H])
    scaled = h2p_ref[...] * jnp.dot(coef, rep2_ref[...],
                                    preferred_element_type=f32)
    self2 = jnp.dot(scaled, sum2_ref[...], preferred_element_type=f32)
    hout = _elu(s2_ref[0] + s2_ref[1] + self2 + b2_ref[...])
    bm = batch_ref[0]
    g = lax.broadcasted_iota(jnp.int32, (NG, bm.shape[1]), 0)
    oh = (bm == g).astype(f32)
    part = jnp.dot(oh, hout, preferred_element_type=f32)
    pc = jnp.sum(oh, axis=1, keepdims=True)

    @pl.when(i == 0)
    def _():
        out_ref[...] = jnp.zeros_like(out_ref)
        cnt_ref[...] = jnp.zeros_like(cnt_ref)

    out_ref[...] += part
    cnt_ref[...] += jnp.broadcast_to(pc, cnt_ref.shape)

    @pl.when(i == pl.num_programs(0) - 1)
    def _():
        out_ref[...] = out_ref[...] / jnp.maximum(cnt_ref[...], 1.0)


def _sc_att_body(N, E, src_ref, dst_ref, as2_ref, ad2_ref, z16_ref, d2_out,
                 w2_out, idx_s, idx_d, asb, adb, wb, den_sh, sem0, sem1):
    c = lax.axis_index("c")
    s = lax.axis_index("s")
    wid = c * NS + s
    epw = E // NW

    @pl.when(s == 0)
    def _():
        pltpu.sync_copy(z16_ref, den_sh)

    plsc.subcore_barrier()

    base0 = wid * epw

    def chunk(i, carry):
        be = base0 + i * C
        pltpu.sync_copy(src_ref.at[pl.ds(be, C)], idx_s)
        pltpu.sync_copy(dst_ref.at[pl.ds(be, C)], idx_d)
        cp0 = pltpu.async_copy(as2_ref.at[idx_s], asb, sem0)
        cp1 = pltpu.async_copy(ad2_ref.at[idx_d], adb, sem1)
        cp0.wait()
        cp1.wait()

        def eloop(cc, carry2):
            e = asb[cc, pl.ds(0, HP)] + adb[cc, pl.ds(0, HP)]
            wb[cc, :] = jnp.exp(jnp.maximum(e, 0.2 * e))
            return carry2

        lax.fori_loop(0, C, eloop, 0)

        pltpu.sync_copy(wb, den_sh.at[idx_d], add=True)
        pltpu.sync_copy(wb, w2_out.at[pl.ds(be, C)])
        return carry

    lax.fori_loop(0, epw // C, chunk, 0)

    plsc.subcore_barrier()
    _copy_shared_out(den_sh, d2_out, c, s, N)


def _sc_agg1_body(N, E, src_ref, dst_ref, w1_ref, h1p_ref, z128_ref,
                  s1_out, idx_s, idx_d, wb, hb, mb, agg_sh, sem0):
    c = lax.axis_index("c")
    s = lax.axis_index("s")
    wid = c * NS + s
    epw = E // NW

    @pl.when(s == 0)
    def _():
        pltpu.sync_copy(z128_ref, agg_sh)

    plsc.subcore_barrier()

    base0 = wid * epw

    def chunk(i, carry):
        be = base0 + i * C
        pltpu.sync_copy(src_ref.at[pl.ds(be, C)], idx_s)
        pltpu.sync_copy(dst_ref.at[pl.ds(be, C)], idx_d)
        pltpu.sync_copy(w1_ref.at[pl.ds(be, C)], wb)
        cp0 = pltpu.async_copy(h1p_ref.at[idx_s], hb, sem0)
        cp0.wait()

        def mloop(cc, carry2):
            wrow = wb[cc, :]
            for hh in range(H):
                mb[cc, pl.ds(hh * HID, HID)] = (
                    wrow[hh] * hb[cc, pl.ds(hh * HID, HID)])
            return carry2

        lax.fori_loop(0, C, mloop, 0)

        pltpu.sync_copy(mb, agg_sh.at[idx_d], add=True)
        return carry

    lax.fori_loop(0, epw // C, chunk, 0)

    plsc.subcore_barrier()
    _copy_shared_out(agg_sh, s1_out, c, s, N)


def _sc_beta_body(N, E, dst_ref, w2_ref, dt_ref, b_out, idx_d, wb, dtb, bb,
                  sem0):
    c = lax.axis_index("c")
    s = lax.axis_index("s")
    wid = c * NS + s
    epw = E // NW
    base0 = wid * epw

    def chunk(i, carry):
        be = base0 + i * C
        pltpu.sync_copy(dst_ref.at[pl.ds(be, C)], idx_d)
        pltpu.sync_copy(w2_ref.at[pl.ds(be, C)], wb)
        cp0 = pltpu.async_copy(dt_ref.at[idx_d], dtb, sem0)
        cp0.wait()

        def bloop(cc, carry2):
            bb[cc, :] = wb[cc, :] / (8.0 * dtb[cc, pl.ds(0, HP)])
            return carry2

        lax.fori_loop(0, C, bloop, 0)

        pltpu.sync_copy(bb, b_out.at[pl.ds(be, C)])
        return carry

    lax.fori_loop(0, epw // C, chunk, 0)


def _sc_agg2_body(N, E, src_ref, dst_ref, beta_ref, h2p_ref, z128_ref,
                  s2_out, idx_s, idx_d, bb, hb, yb, agg_sh, sem0):
    c = lax.axis_index("c")
    s = lax.axis_index("s")
    wid = c * NS + s
    epw = E // NW

    @pl.when(s == 0)
    def _():
        pltpu.sync_copy(z128_ref, agg_sh)

    plsc.subcore_barrier()

    base0 = wid * epw

    def do_chunk(be, sz):
        isl = idx_s.at[pl.ds(0, sz)]
        idl = idx_d.at[pl.ds(0, sz)]
        pltpu.sync_copy(src_ref.at[pl.ds(be, sz)], isl)
        pltpu.sync_copy(dst_ref.at[pl.ds(be, sz)], idl)
        pltpu.sync_copy(beta_ref.at[pl.ds(be, sz)], bb.at[pl.ds(0, sz)])
        cp1 = pltpu.async_copy(h2p_ref.at[isl], hb.at[pl.ds(0, sz)], sem0)
        cp1.wait()

        def yloop(cc, carry2):
            brow = bb[cc, :]
            bs = [brow[hh] for hh in range(H)]
            for j in range(H):
                acc = bs[0] * hb[cc, pl.ds(j * 16, 16)]
                for hh in range(1, H):
                    acc = acc + bs[hh] * hb[cc, pl.ds(hh * OUTC + j * 16, 16)]
                yb[cc, pl.ds(j * 16, 16)] = acc
            return carry2

        lax.fori_loop(0, sz, yloop, 0)
        pltpu.sync_copy(yb.at[pl.ds(0, sz)], agg_sh.at[idl], add=True)

    nfull = epw // C2
    rem = epw - nfull * C2

    def chunk(i, carry):
        do_chunk(base0 + i * C2, C2)
        return carry

    lax.fori_loop(0, nfull, chunk, 0)
    if rem:
        do_chunk(base0 + nfull * C2, rem)

    plsc.subcore_barrier()
    _copy_shared_out(agg_sh, s2_out, c, s, N)


def kernel(x, edge_index, batch, W1, a_src1, a_dst1, b1, W2, a_src2, a_dst2,
           b2):
    N, IN = x.shape
    E = edge_index.shape[1]
    F1 = H * HID
    F2 = H * OUTC
    assert E % (NW * C) == 0 and N % NS == 0

    src = edge_index[0].astype(jnp.int32)
    dst = edge_index[1].astype(jnp.int32)

    FW = 128

    def padh(m):
        return jnp.pad(m, ((0, 0), (0, FW - H)))

    A1s = padh((jnp.eye(H, dtype=f32)[:, None, :] *
                a_src1[:, :, None]).reshape(F1, H))
    A1d = padh((jnp.eye(H, dtype=f32)[:, None, :] *
                a_dst1[:, :, None]).reshape(F1, H))
    W2r = W2.reshape(F1, H, OUTC)
    Ws2 = padh(jnp.einsum('khj,hj->kh', W2r, a_src2))
    Wd2 = padh(jnp.einsum('khj,hj->kh', W2r, a_dst2))
    hh = jnp.arange(H, dtype=jnp.int32)
    REP = (hh[:, None] == (jnp.arange(F1) // HID)[None, :]).astype(f32)
    REP2 = (hh[:, None] == (jnp.arange(F2) // OUTC)[None, :]).astype(f32)
    SUM2 = ((jnp.arange(F2) % OUTC)[:, None] ==
            jnp.arange(OUTC)[None, :]).astype(f32)
    b1m = b1.reshape(1, F1)
    b2m = b2.reshape(1, OUTC)
    z16 = jnp.zeros((N, HP), f32)
    z128 = jnp.zeros((N, OUTC), f32)

    BN = 1000
    GB = N // BN
    batch3 = batch.astype(jnp.int32).reshape(GB, 1, BN)

    h1p, as1, ad1 = pl.pallas_call(
        _k0_body,
        grid=(GB,),
        in_specs=[
            pl.BlockSpec((BN, IN), lambda i: (i, 0)),
            pl.BlockSpec((IN, F1), lambda i: (0, 0)),
            pl.BlockSpec((F1, FW), lambda i: (0, 0)),
            pl.BlockSpec((F1, FW), lambda i: (0, 0)),
        ],
        out_specs=[
            pl.BlockSpec((BN, F1), lambda i: (i, 0)),
            pl.BlockSpec((BN, FW), lambda i: (i, 0)),
            pl.BlockSpec((BN, FW), lambda i: (i, 0)),
        ],
        out_shape=[
            jax.ShapeDtypeStruct((N, F1), f32),
            jax.ShapeDtypeStruct((N, FW), f32),
            jax.ShapeDtypeStruct((N, FW), f32),
        ],
    )(x, W1, A1s, A1d)

    mesh = plsc.VectorSubcoreMesh(core_axis_name="c", subcore_axis_name="s")
    att_scratch = [
        pltpu.VMEM((C,), jnp.int32),
        pltpu.VMEM((C,), jnp.int32),
        pltpu.VMEM((C, FW), f32),
        pltpu.VMEM((C, FW), f32),
        pltpu.VMEM((C, HP), f32),
        pltpu.VMEM_SHARED((N, HP), f32),
        pltpu.SemaphoreType.DMA,
        pltpu.SemaphoreType.DMA,
    ]
    att_out = [
        jax.ShapeDtypeStruct((NC, N, HP), f32),
        jax.ShapeDtypeStruct((E, HP), f32),
    ]
    d1, w1 = pl.kernel(
        functools.partial(_sc_att_body, N, E),
        out_type=att_out,
        mesh=mesh,
        scratch_types=att_scratch,
    )(src, dst, as1, ad1, z16)

    s1 = pl.kernel(
        functools.partial(_sc_agg1_body, N, E),
        out_type=jax.ShapeDtypeStruct((NC, N, F1), f32),
        mesh=mesh,
        scratch_types=[
            pltpu.VMEM((C,), jnp.int32),
            pltpu.VMEM((C,), jnp.int32),
            pltpu.VMEM((C, HP), f32),
            pltpu.VMEM((C, F1), f32),
            pltpu.VMEM((C, F1), f32),
            pltpu.VMEM_SHARED((N, F1), f32),
            pltpu.SemaphoreType.DMA,
        ],
    )(src, dst, w1, h1p, z128)

    h2p, as2, ad2 = pl.pallas_call(
        _k1_body,
        grid=(GB,),
        in_specs=[
            pl.BlockSpec((NC, BN, F1), lambda i: (0, i, 0)),
            pl.BlockSpec((NC, BN, HP), lambda i: (0, i, 0)),
            pl.BlockSpec((BN, F1), lambda i: (i, 0)),
            pl.BlockSpec((BN, FW), lambda i: (i, 0)),
            pl.BlockSpec((BN, FW), lambda i: (i, 0)),
            pl.BlockSpec((1, F1), lambda i: (0, 0)),
            pl.BlockSpec((F1, F2), lambda i: (0, 0)),
            pl.BlockSpec((F1, FW), lambda i: (0, 0)),
            pl.BlockSpec((F1, FW), lambda i: (0, 0)),
            pl.BlockSpec((H, F1), lambda i: (0, 0)),
        ],
        out_specs=[
            pl.BlockSpec((BN, F2), lambda i: (i, 0)),
            pl.BlockSpec((BN, FW), lambda i: (i, 0)),
            pl.BlockSpec((BN, FW), lambda i: (i, 0)),
        ],
        out_shape=[
            jax.ShapeDtypeStruct((N, F2), f32),
            jax.ShapeDtypeStruct((N, FW), f32),
            jax.ShapeDtypeStruct((N, FW), f32),
        ],
    )(s1, d1, h1p, as1, ad1, b1m, W2, Ws2, Wd2, REP)

    d2, w2 = pl.kernel(
        functools.partial(_sc_att_body, N, E),
        out_type=att_out,
        mesh=mesh,
        scratch_types=att_scratch,
    )(src, dst, as2, ad2, z16)

    dt = pl.pallas_call(
        _k2_body,
        out_shape=jax.ShapeDtypeStruct((N, FW), f32),
    )(as2, ad2, d2)

    beta = pl.kernel(
        functools.partial(_sc_beta_body, N, E),
        out_type=jax.ShapeDtypeStruct((E, HP), f32),
        mesh=mesh,
        scratch_types=[
            pltpu.VMEM((C,), jnp.int32),
            pltpu.VMEM((C, HP), f32),
            pltpu.VMEM((C, FW), f32),
            pltpu.VMEM((C, HP), f32),
            pltpu.SemaphoreType.DMA,
        ],
    )(dst, w2, dt)

    s2 = pl.kernel(
        functools.partial(_sc_agg2_body, N, E),
        out_type=jax.ShapeDtypeStruct((NC, N, OUTC), f32),
        mesh=mesh,
        scratch_types=[
            pltpu.VMEM((C2,), jnp.int32),
            pltpu.VMEM((C2,), jnp.int32),
            pltpu.VMEM((C2, HP), f32),
            pltpu.VMEM((C2, F2), f32),
            pltpu.VMEM((C2, OUTC), f32),
            pltpu.VMEM_SHARED((N, OUTC), f32),
            pltpu.SemaphoreType.DMA,
        ],
    )(src, dst, beta, h2p, z128)

    out = pl.pallas_call(
        _k3_body,
        grid=(GB,),
        in_specs=[
            pl.BlockSpec((NC, BN, OUTC), lambda i: (0, i, 0)),
            pl.BlockSpec((BN, F2), lambda i: (i, 0)),
            pl.BlockSpec((BN, FW), lambda i: (i, 0)),
            pl.BlockSpec((BN, FW), lambda i: (i, 0)),
            pl.BlockSpec((BN, FW), lambda i: (i, 0)),
            pl.BlockSpec((1, OUTC), lambda i: (0, 0)),
            pl.BlockSpec((1, 1, BN), lambda i: (i, 0, 0)),
            pl.BlockSpec((H, F2), lambda i: (0, 0)),
            pl.BlockSpec((F2, OUTC), lambda i: (0, 0)),
        ],
        out_specs=pl.BlockSpec((NG, OUTC), lambda i: (0, 0)),
        out_shape=jax.ShapeDtypeStruct((NG, OUTC), f32),
        scratch_shapes=[pltpu.VMEM((NG, OUTC), f32)],
    )(s2, h2p, as2, ad2, dt, b2m, batch3, REP2, SUM2)

    return out


# speedup vs baseline: 24.9973x; 1.2025x over previous
import functools
import jax
import jax.numpy as jnp
from jax import lax
from jax.experimental import pallas as pl
from jax.experimental.pallas import tpu as pltpu
import jax.experimental.pallas.tpu_sc as plsc

H = 8
HP = 16
HID = 16
OUTC = 128
NG = 128
NC = 2
NS = 16
NW = NC * NS
C = 80
C2 = 32

f32 = jnp.float32


def _lrelu(v):
    return jnp.maximum(v, 0.2 * v)


def _elu(v):
    return jnp.where(v > 0, v, jnp.exp(v) - 1.0)


def _copy_shared_out(sh_ref, out_ref, c, s, N):
    nm = (N // NS) & ~7
    tail = N - NS * nm
    off = s * nm
    pltpu.sync_copy(sh_ref.at[pl.ds(off, nm)], out_ref.at[c, pl.ds(off, nm)])
    if tail:
        @pl.when(s == NS - 1)
        def _():
            pltpu.sync_copy(sh_ref.at[pl.ds(NS * nm, tail)],
                            out_ref.at[c, pl.ds(NS * nm, tail)])


def _k0_body(x_ref, w1_ref, a1s_ref, a1d_ref, h1p_ref, as1_ref, ad1_ref):
    h = jnp.dot(x_ref[...], w1_ref[...], preferred_element_type=f32)
    h1p_ref[...] = h
    as1_ref[...] = jnp.dot(h, a1s_ref[...], preferred_element_type=f32)
    ad1_ref[...] = jnp.dot(h, a1d_ref[...], preferred_element_type=f32)


def _k1_body(s1_ref, d1_ref, h1p_ref, as1_ref, ad1_ref, b1_ref, w2_ref,
             ws2_ref, wd2_ref, rep_ref, h2p_ref, as2_ref, ad2_ref):
    wl1 = jnp.exp(_lrelu((as1_ref[...] + ad1_ref[...])[:, :H]))
    den8 = d1_ref[0][:, :H] + d1_ref[1][:, :H] + wl1
    den = jnp.dot(den8, rep_ref[...], preferred_element_type=f32)
    num = s1_ref[0] + s1_ref[1] + jnp.dot(
        wl1, rep_ref[...], preferred_element_type=f32) * h1p_ref[...]
    h1 = _elu(num / den + b1_ref[...])
    h2p_ref[...] = jnp.dot(h1, w2_ref[...], preferred_element_type=f32)
    as2_ref[...] = jnp.dot(h1, ws2_ref[...], preferred_element_type=f32)
    ad2_ref[...] = jnp.dot(h1, wd2_ref[...], preferred_element_type=f32)


def _k2_body(as2_ref, ad2_ref, d2_ref, dt_ref):
    wl2 = jnp.exp(_lrelu(as2_ref[...] + ad2_ref[...]))
    dsum = d2_ref[0] + d2_ref[1]
    pad = jnp.zeros((dsum.shape[0], wl2.shape[1] - HP), f32)
    dt_ref[...] = wl2 + jnp.concatenate([dsum, pad], axis=1)


def _k3_body(s2_ref, h2p_ref, as2_ref, ad2_ref, dt_ref, b2_ref, batch_ref,
             rep2_ref, sum2_ref, out_ref, cnt_ref):
    i = pl.program_id(0)
    wl2 = jnp.exp(_lrelu((as2_ref[...] + ad2_ref[...])[:, :H]))
    coef = wl2 / (8.0 * dt_ref[...][:, :H])
    scaled = h2p_ref[...] * jnp.dot(coef, rep2_ref[...],
                                    preferred_element_type=f32)
    self2 = jnp.dot(scaled, sum2_ref[...], preferred_element_type=f32)
    hout = _elu(s2_ref[0] + s2_ref[1] + self2 + b2_ref[...])
    bm = batch_ref[0]
    g = lax.broadcasted_iota(jnp.int32, (NG, bm.shape[1]), 0)
    oh = (bm == g).astype(f32)
    part = jnp.dot(oh, hout, preferred_element_type=f32)
    pc = jnp.sum(oh, axis=1, keepdims=True)

    @pl.when(i == 0)
    def _():
        out_ref[...] = jnp.zeros_like(out_ref)
        cnt_ref[...] = jnp.zeros_like(cnt_ref)

    out_ref[...] += part
    cnt_ref[...] += jnp.broadcast_to(pc, cnt_ref.shape)

    @pl.when(i == pl.num_programs(0) - 1)
    def _():
        out_ref[...] = out_ref[...] / jnp.maximum(cnt_ref[...], 1.0)


def _sc_att_body(N, E, src_ref, dst_ref, as2_ref, ad2_ref, z16_ref, d2_out,
                 w2_out, idx_s, idx_d, asb, adb, wb, den_sh, sem0, sem1):
    c = lax.axis_index("c")
    s = lax.axis_index("s")
    wid = c * NS + s
    epw = E // NW

    @pl.when(s == 0)
    def _():
        pltpu.sync_copy(z16_ref, den_sh)

    plsc.subcore_barrier()

    base0 = wid * epw

    def chunk(i, carry):
        be = base0 + i * C
        pltpu.sync_copy(src_ref.at[pl.ds(be, C)], idx_s)
        pltpu.sync_copy(dst_ref.at[pl.ds(be, C)], idx_d)
        cp0 = pltpu.async_copy(as2_ref.at[idx_s], asb, sem0)
        cp1 = pltpu.async_copy(ad2_ref.at[idx_d], adb, sem1)
        cp0.wait()
        cp1.wait()

        @plsc.parallel_loop(0, C, unroll=4)
        def eloop(cc):
            e = asb[cc, pl.ds(0, HP)] + adb[cc, pl.ds(0, HP)]
            wb[cc, :] = jnp.exp(jnp.maximum(e, 0.2 * e))

        pltpu.sync_copy(wb, den_sh.at[idx_d], add=True)
        pltpu.sync_copy(wb, w2_out.at[pl.ds(be, C)])
        return carry

    lax.fori_loop(0, epw // C, chunk, 0)

    plsc.subcore_barrier()
    _copy_shared_out(den_sh, d2_out, c, s, N)


def _sc_agg1_body(N, E, src_ref, dst_ref, w1_ref, h1p_ref, z128_ref,
                  s1_out, idx_s, idx_d, wb, hb, mb, agg_sh, sem0):
    c = lax.axis_index("c")
    s = lax.axis_index("s")
    wid = c * NS + s
    epw = E // NW

    @pl.when(s == 0)
    def _():
        pltpu.sync_copy(z128_ref, agg_sh)

    plsc.subcore_barrier()

    base0 = wid * epw

    def chunk(i, carry):
        be = base0 + i * C
        pltpu.sync_copy(src_ref.at[pl.ds(be, C)], idx_s)
        pltpu.sync_copy(dst_ref.at[pl.ds(be, C)], idx_d)
        pltpu.sync_copy(w1_ref.at[pl.ds(be, C)], wb)
        cp0 = pltpu.async_copy(h1p_ref.at[idx_s], hb, sem0)
        cp0.wait()

        @plsc.parallel_loop(0, C, unroll=2)
        def mloop(cc):
            wrow = wb[cc, :]
            for hh in range(H):
                mb[cc, pl.ds(hh * HID, HID)] = (
                    wrow[hh] * hb[cc, pl.ds(hh * HID, HID)])

        pltpu.sync_copy(mb, agg_sh.at[idx_d], add=True)
        return carry

    lax.fori_loop(0, epw // C, chunk, 0)

    plsc.subcore_barrier()
    _copy_shared_out(agg_sh, s1_out, c, s, N)


def _sc_beta_body(N, E, dst_ref, w2_ref, dt_ref, b_out, idx_d, wb, dtb, bb,
                  sem0):
    c = lax.axis_index("c")
    s = lax.axis_index("s")
    wid = c * NS + s
    epw = E // NW
    base0 = wid * epw

    def chunk(i, carry):
        be = base0 + i * C
        pltpu.sync_copy(dst_ref.at[pl.ds(be, C)], idx_d)
        pltpu.sync_copy(w2_ref.at[pl.ds(be, C)], wb)
        cp0 = pltpu.async_copy(dt_ref.at[idx_d], dtb, sem0)
        cp0.wait()

        @plsc.parallel_loop(0, C, unroll=4)
        def bloop(cc):
            bb[cc, :] = wb[cc, :] / (8.0 * dtb[cc, pl.ds(0, HP)])

        pltpu.sync_copy(bb, b_out.at[pl.ds(be, C)])
        return carry

    lax.fori_loop(0, epw // C, chunk, 0)


def _sc_agg2_body(N, E, src_ref, dst_ref, beta_ref, h2p_ref, z128_ref,
                  s2_out, idx_s, idx_d, bb, hb, yb, agg_sh, sem0):
    c = lax.axis_index("c")
    s = lax.axis_index("s")
    wid = c * NS + s
    epw = E // NW

    @pl.when(s == 0)
    def _():
        pltpu.sync_copy(z128_ref, agg_sh)

    plsc.subcore_barrier()

    base0 = wid * epw

    def do_chunk(be, sz):
        isl = idx_s.at[pl.ds(0, sz)]
        idl = idx_d.at[pl.ds(0, sz)]
        pltpu.sync_copy(src_ref.at[pl.ds(be, sz)], isl)
        pltpu.sync_copy(dst_ref.at[pl.ds(be, sz)], idl)
        pltpu.sync_copy(beta_ref.at[pl.ds(be, sz)], bb.at[pl.ds(0, sz)])
        cp1 = pltpu.async_copy(h2p_ref.at[isl], hb.at[pl.ds(0, sz)], sem0)
        cp1.wait()

        @plsc.parallel_loop(0, sz, unroll=2)
        def yloop(cc):
            brow = bb[cc, :]
            bs = [brow[hh] for hh in range(H)]
            for j in range(H):
                acc = bs[0] * hb[cc, pl.ds(j * 16, 16)]
                for hh in range(1, H):
                    acc = acc + bs[hh] * hb[cc, pl.ds(hh * OUTC + j * 16, 16)]
                yb[cc, pl.ds(j * 16, 16)] = acc
        pltpu.sync_copy(yb.at[pl.ds(0, sz)], agg_sh.at[idl], add=True)

    nfull = epw // C2
    rem = epw - nfull * C2

    def chunk(i, carry):
        do_chunk(base0 + i * C2, C2)
        return carry

    lax.fori_loop(0, nfull, chunk, 0)
    if rem:
        do_chunk(base0 + nfull * C2, rem)

    plsc.subcore_barrier()
    _copy_shared_out(agg_sh, s2_out, c, s, N)


def kernel(x, edge_index, batch, W1, a_src1, a_dst1, b1, W2, a_src2, a_dst2,
           b2):
    N, IN = x.shape
    E = edge_index.shape[1]
    F1 = H * HID
    F2 = H * OUTC
    assert E % (NW * C) == 0 and N % NS == 0

    src = edge_index[0].astype(jnp.int32)
    dst = edge_index[1].astype(jnp.int32)

    FW = 128

    def padh(m):
        return jnp.pad(m, ((0, 0), (0, FW - H)))

    A1s = padh((jnp.eye(H, dtype=f32)[:, None, :] *
                a_src1[:, :, None]).reshape(F1, H))
    A1d = padh((jnp.eye(H, dtype=f32)[:, None, :] *
                a_dst1[:, :, None]).reshape(F1, H))
    W2r = W2.reshape(F1, H, OUTC)
    Ws2 = padh(jnp.einsum('khj,hj->kh', W2r, a_src2))
    Wd2 = padh(jnp.einsum('khj,hj->kh', W2r, a_dst2))
    hh = jnp.arange(H, dtype=jnp.int32)
    REP = (hh[:, None] == (jnp.arange(F1) // HID)[None, :]).astype(f32)
    REP2 = (hh[:, None] == (jnp.arange(F2) // OUTC)[None, :]).astype(f32)
    SUM2 = ((jnp.arange(F2) % OUTC)[:, None] ==
            jnp.arange(OUTC)[None, :]).astype(f32)
    b1m = b1.reshape(1, F1)
    b2m = b2.reshape(1, OUTC)
    z16 = jnp.zeros((N, HP), f32)
    z128 = jnp.zeros((N, OUTC), f32)

    BN = 1000
    GB = N // BN
    batch3 = batch.astype(jnp.int32).reshape(GB, 1, BN)

    h1p, as1, ad1 = pl.pallas_call(
        _k0_body,
        grid=(GB,),
        in_specs=[
            pl.BlockSpec((BN, IN), lambda i: (i, 0)),
            pl.BlockSpec((IN, F1), lambda i: (0, 0)),
            pl.BlockSpec((F1, FW), lambda i: (0, 0)),
            pl.BlockSpec((F1, FW), lambda i: (0, 0)),
        ],
        out_specs=[
            pl.BlockSpec((BN, F1), lambda i: (i, 0)),
            pl.BlockSpec((BN, FW), lambda i: (i, 0)),
            pl.BlockSpec((BN, FW), lambda i: (i, 0)),
        ],
        out_shape=[
            jax.ShapeDtypeStruct((N, F1), f32),
            jax.ShapeDtypeStruct((N, FW), f32),
            jax.ShapeDtypeStruct((N, FW), f32),
        ],
    )(x, W1, A1s, A1d)

    mesh = plsc.VectorSubcoreMesh(core_axis_name="c", subcore_axis_name="s")
    att_scratch = [
        pltpu.VMEM((C,), jnp.int32),
        pltpu.VMEM((C,), jnp.int32),
        pltpu.VMEM((C, FW), f32),
        pltpu.VMEM((C, FW), f32),
        pltpu.VMEM((C, HP), f32),
        pltpu.VMEM_SHARED((N, HP), f32),
        pltpu.SemaphoreType.DMA,
        pltpu.SemaphoreType.DMA,
    ]
    att_out = [
        jax.ShapeDtypeStruct((NC, N, HP), f32),
        jax.ShapeDtypeStruct((E, HP), f32),
    ]
    d1, w1 = pl.kernel(
        functools.partial(_sc_att_body, N, E),
        out_type=att_out,
        mesh=mesh,
        scratch_types=att_scratch,
    )(src, dst, as1, ad1, z16)

    s1 = pl.kernel(
        functools.partial(_sc_agg1_body, N, E),
        out_type=jax.ShapeDtypeStruct((NC, N, F1), f32),
        mesh=mesh,
        scratch_types=[
            pltpu.VMEM((C,), jnp.int32),
            pltpu.VMEM((C,), jnp.int32),
            pltpu.VMEM((C, HP), f32),
            pltpu.VMEM((C, F1), f32),
            pltpu.VMEM((C, F1), f32),
            pltpu.VMEM_SHARED((N, F1), f32),
            pltpu.SemaphoreType.DMA,
        ],
    )(src, dst, w1, h1p, z128)

    h2p, as2, ad2 = pl.pallas_call(
        _k1_body,
        grid=(GB,),
        in_specs=[
            pl.BlockSpec((NC, BN, F1), lambda i: (0, i, 0)),
            pl.BlockSpec((NC, BN, HP), lambda i: (0, i, 0)),
            pl.BlockSpec((BN, F1), lambda i: (i, 0)),
            pl.BlockSpec((BN, FW), lambda i: (i, 0)),
            pl.BlockSpec((BN, FW), lambda i: (i, 0)),
            pl.BlockSpec((1, F1), lambda i: (0, 0)),
            pl.BlockSpec((F1, F2), lambda i: (0, 0)),
            pl.BlockSpec((F1, FW), lambda i: (0, 0)),
            pl.BlockSpec((F1, FW), lambda i: (0, 0)),
            pl.BlockSpec((H, F1), lambda i: (0, 0)),
        ],
        out_specs=[
            pl.BlockSpec((BN, F2), lambda i: (i, 0)),
            pl.BlockSpec((BN, FW), lambda i: (i, 0)),
            pl.BlockSpec((BN, FW), lambda i: (i, 0)),
        ],
        out_shape=[
            jax.ShapeDtypeStruct((N, F2), f32),
            jax.ShapeDtypeStruct((N, FW), f32),
            jax.ShapeDtypeStruct((N, FW), f32),
        ],
    )(s1, d1, h1p, as1, ad1, b1m, W2, Ws2, Wd2, REP)

    d2, w2 = pl.kernel(
        functools.partial(_sc_att_body, N, E),
        out_type=att_out,
        mesh=mesh,
        scratch_types=att_scratch,
    )(src, dst, as2, ad2, z16)

    dt = pl.pallas_call(
        _k2_body,
        out_shape=jax.ShapeDtypeStruct((N, FW), f32),
    )(as2, ad2, d2)

    beta = pl.kernel(
        functools.partial(_sc_beta_body, N, E),
        out_type=jax.ShapeDtypeStruct((E, HP), f32),
        mesh=mesh,
        scratch_types=[
            pltpu.VMEM((C,), jnp.int32),
            pltpu.VMEM((C, HP), f32),
            pltpu.VMEM((C, FW), f32),
            pltpu.VMEM((C, HP), f32),
            pltpu.SemaphoreType.DMA,
        ],
    )(dst, w2, dt)

    s2 = pl.kernel(
        functools.partial(_sc_agg2_body, N, E),
        out_type=jax.ShapeDtypeStruct((NC, N, OUTC), f32),
        mesh=mesh,
        scratch_types=[
            pltpu.VMEM((C2,), jnp.int32),
            pltpu.VMEM((C2,), jnp.int32),
            pltpu.VMEM((C2, HP), f32),
            pltpu.VMEM((C2, F2), f32),
            pltpu.VMEM((C2, OUTC), f32),
            pltpu.VMEM_SHARED((N, OUTC), f32),
            pltpu.SemaphoreType.DMA,
        ],
    )(src, dst, beta, h2p, z128)

    out = pl.pallas_call(
        _k3_body,
        grid=(GB,),
        in_specs=[
            pl.BlockSpec((NC, BN, OUTC), lambda i: (0, i, 0)),
            pl.BlockSpec((BN, F2), lambda i: (i, 0)),
            pl.BlockSpec((BN, FW), lambda i: (i, 0)),
            pl.BlockSpec((BN, FW), lambda i: (i, 0)),
            pl.BlockSpec((BN, FW), lambda i: (i, 0)),
            pl.BlockSpec((1, OUTC), lambda i: (0, 0)),
            pl.BlockSpec((1, 1, BN), lambda i: (i, 0, 0)),
            pl.BlockSpec((H, F2), lambda i: (0, 0)),
            pl.BlockSpec((F2, OUTC), lambda i: (0, 0)),
        ],
        out_specs=pl.BlockSpec((NG, OUTC), lambda i: (0, 0)),
        out_shape=jax.ShapeDtypeStruct((NG, OUTC), f32),
        scratch_shapes=[pltpu.VMEM((NG, OUTC), f32)],
    )(s2, h2p, as2, ad2, dt, b2m, batch3, REP2, SUM2)

    return out
